# baseline (speedup 1.0000x reference)
import jax
import jax.numpy as jnp
from jax.experimental import pallas as pl
from jax.experimental.pallas import tpu as pltpu

_TOPK = 32
_DBLK = 128
_CHUNK = 256
_HEAD = 4


def _proj_kernel(x_ref, w_ref, o_ref):
    o_ref[...] = jnp.dot(x_ref[...], w_ref[...],
                         preferred_element_type=jnp.float32)


def _simtopk_kernel(hd_ref, htp_ref, w1_ref, bmat_ref, tri_ref,
                    upre_ref, res_ref, flt_ref):
    hd = jnp.dot(hd_ref[...], w1_ref[...],
                 preferred_element_type=jnp.float32)
    logits = jax.lax.dot_general(
        hd, htp_ref[...], (((1,), (1,)), ((), ())),
        preferred_element_type=jnp.float32)
    s = jax.nn.sigmoid(logits)
    res_ref[...] = s
    d, t_num = s.shape
    nc = t_num // _CHUNK
    head = min(_HEAD, nc)
    one = jnp.float32(1.0)
    tri = tri_ref[...]

    eqb1 = (s == one).astype(jnp.bfloat16)
    tot1 = jax.lax.dot_general(
        eqb1, bmat_ref[...], (((1,), (0,)), ((), ())),
        preferred_element_type=jnp.float32)
    cnt1 = jnp.sum(tot1, axis=1, keepdims=True)
    fast = jnp.all(cnt1 >= _TOPK)

    @pl.when(fast)
    def _():
        pre = jax.lax.dot_general(
            tot1.astype(jnp.bfloat16), upre_ref[...],
            (((1,), (0,)), ((), ())), preferred_element_type=jnp.float32)
        needc_all = jnp.float32(_TOPK) - pre

        def chunk_mask_fast(c):
            sl = s[:, c * _CHUNK:(c + 1) * _CHUNK]
            eqc = (sl == one)
            pref = jax.lax.dot_general(
                eqc.astype(jnp.bfloat16), tri, (((1,), (0,)), ((), ())),
                preferred_element_type=jnp.float32)
            keep = eqc & (pref <= needc_all[:, c:c + 1])
            flt_ref[:, c * _CHUNK:(c + 1) * _CHUNK] = jnp.where(
                keep, sl, jnp.float32(0.0))

        for c in range(head):
            chunk_mask_fast(c)

        if head < nc:
            tail_zero = jnp.max(needc_all[:, head:head + 1]) < 1.0

            @pl.when(tail_zero)
            def _():
                flt_ref[:, head * _CHUNK:] = jnp.zeros(
                    (d, t_num - head * _CHUNK), jnp.float32)

            @pl.when(jnp.logical_not(tail_zero))
            def _():
                for c in range(head, nc):
                    chunk_mask_fast(c)

    @pl.when(jnp.logical_not(fast))
    def _():
        hi = jnp.max(s, axis=1, keepdims=True)
        key = jax.lax.bitcast_convert_type(s, jnp.int32)
        hik = jax.lax.bitcast_convert_type(hi, jnp.int32)
        lok = jnp.zeros_like(hik)

        def body(_, carry):
            lo, h = carry
            mid = (lo + h + 1) >> 1
            cnt = jnp.sum((key >= mid).astype(jnp.int32), axis=1,
                          keepdims=True)
            ok = cnt >= _TOPK
            return jnp.where(ok, mid, lo), jnp.where(ok, h, mid - 1)

        lok, _hik = jax.lax.fori_loop(0, 31, body, (lok, hik))
        t = jax.lax.bitcast_convert_type(lok, jnp.float32)
        eqb = (s == t).astype(jnp.bfloat16)
        tot = jax.lax.dot_general(
            eqb, bmat_ref[...], (((1,), (0,)), ((), ())),
            preferred_element_type=jnp.float32)
        gt_cnt = jnp.sum((s > t).astype(jnp.float32), axis=1, keepdims=True)
        pre = jax.lax.dot_general(
            tot.astype(jnp.bfloat16), upre_ref[...],
            (((1,), (0,)), ((), ())), preferred_element_type=jnp.float32)
        needc_all = (_TOPK - gt_cnt) - pre

        for c in range(nc):
            sl = s[:, c * _CHUNK:(c + 1) * _CHUNK]
            eqc = (sl == t)
            pref = jax.lax.dot_general(
                eqc.astype(jnp.bfloat16), tri, (((1,), (0,)), ((), ())),
                preferred_element_type=jnp.float32)
            keep = (sl > t) | (eqc & (pref <= needc_all[:, c:c + 1]))
            flt_ref[:, c * _CHUNK:(c + 1) * _CHUNK] = jnp.where(
                keep, sl, jnp.float32(0.0))


def kernel(H_d, H_t, W1, W2):
    d_num, d_dim = H_d.shape
    t_num, t_dim = H_t.shape
    units = W1.shape[1]
    nc = t_num // _CHUNK
    blk = min(1024, t_num)
    Ht = pl.pallas_call(
        _proj_kernel,
        grid=(t_num // blk,),
        in_specs=[
            pl.BlockSpec((blk, t_dim), lambda i: (i, 0)),
            pl.BlockSpec((t_dim, units), lambda i: (0, 0)),
        ],
        out_specs=pl.BlockSpec((blk, units), lambda i: (i, 0)),
        out_shape=jax.ShapeDtypeStruct((t_num, units), jnp.float32),
        compiler_params=pltpu.CompilerParams(
            dimension_semantics=("parallel",)),
    )(H_t, W2)
    col = jnp.arange(t_num, dtype=jnp.int32)
    bmat = (col[:, None] // _CHUNK
            == jnp.arange(nc, dtype=jnp.int32)[None, :]).astype(jnp.bfloat16)
    r256 = jnp.arange(_CHUNK, dtype=jnp.int32)
    tri = (r256[:, None] <= r256[None, :]).astype(jnp.bfloat16)
    rnc = jnp.arange(nc, dtype=jnp.int32)
    upre = (rnc[:, None] < rnc[None, :]).astype(jnp.bfloat16)

    res, flt = pl.pallas_call(
        _simtopk_kernel,
        grid=(d_num // _DBLK,),
        in_specs=[
            pl.BlockSpec((_DBLK, d_dim), lambda i: (i, 0)),
            pl.BlockSpec((t_num, units), lambda i: (0, 0)),
            pl.BlockSpec((d_dim, units), lambda i: (0, 0)),
            pl.BlockSpec((t_num, nc), lambda i: (0, 0)),
            pl.BlockSpec((_CHUNK, _CHUNK), lambda i: (0, 0)),
            pl.BlockSpec((nc, nc), lambda i: (0, 0)),
        ],
        out_specs=[
            pl.BlockSpec((_DBLK, t_num), lambda i: (i, 0)),
            pl.BlockSpec((_DBLK, t_num), lambda i: (i, 0)),
        ],
        out_shape=[
            jax.ShapeDtypeStruct((d_num, t_num), jnp.float32),
            jax.ShapeDtypeStruct((d_num, t_num), jnp.float32),
        ],
        compiler_params=pltpu.CompilerParams(
            dimension_semantics=("arbitrary",)),
    )(H_d, Ht, W1, bmat, tri, upre)
    return res, flt


# speedup vs baseline: 1.0863x; 1.0863x over previous
import jax
import jax.numpy as jnp
from jax.experimental import pallas as pl
from jax.experimental.pallas import tpu as pltpu

_TOPK = 32
_DBLK = 256
_CHUNK = 256
_HEAD = 4


def _proj_kernel(x_ref, w_ref, o_ref):
    o_ref[...] = jnp.dot(x_ref[...], w_ref[...],
                         preferred_element_type=jnp.float32)


def _simtopk_kernel(hd_ref, htp_ref, w1_ref, tri_ref,
                    upre_ref, res_ref, flt_ref):
    hd = jnp.dot(hd_ref[...], w1_ref[...],
                 preferred_element_type=jnp.float32)
    logits = jax.lax.dot_general(
        hd, htp_ref[...], (((1,), (1,)), ((), ())),
        preferred_element_type=jnp.float32)
    s = jax.nn.sigmoid(logits)
    res_ref[...] = s
    d, t_num = s.shape
    nc = t_num // _CHUNK
    head = min(_HEAD, nc)
    one = jnp.float32(1.0)
    tri = tri_ref[...]

    _TS = 2048
    tot1 = jnp.zeros((d, nc), jnp.float32)
    for j in range(0, t_num, _TS):
        w = min(_TS, t_num - j)
        eqs = (s[:, j:j + w] == one).astype(jnp.bfloat16)
        bsl = (jax.lax.broadcasted_iota(jnp.int32, (w, nc), 0) // _CHUNK
               + j // _CHUNK
               == jax.lax.broadcasted_iota(jnp.int32, (w, nc), 1)
               ).astype(jnp.bfloat16)
        tot1 = tot1 + jax.lax.dot_general(
            eqs, bsl, (((1,), (0,)), ((), ())),
            preferred_element_type=jnp.float32)
    cnt1 = jnp.sum(tot1, axis=1, keepdims=True)
    fast = jnp.all(cnt1 >= _TOPK)

    @pl.when(fast)
    def _():
        pre = jax.lax.dot_general(
            tot1.astype(jnp.bfloat16), upre_ref[...],
            (((1,), (0,)), ((), ())), preferred_element_type=jnp.float32)
        needc_all = jnp.float32(_TOPK) - pre

        def chunk_mask_fast(c):
            sl = s[:, c * _CHUNK:(c + 1) * _CHUNK]
            eqc = (sl == one)
            pref = jax.lax.dot_general(
                eqc.astype(jnp.bfloat16), tri, (((1,), (0,)), ((), ())),
                preferred_element_type=jnp.float32)
            keep = eqc & (pref <= needc_all[:, c:c + 1])
            flt_ref[:, c * _CHUNK:(c + 1) * _CHUNK] = jnp.where(
                keep, sl, jnp.float32(0.0))

        for c in range(head):
            chunk_mask_fast(c)

        if head < nc:
            tail_zero = jnp.max(needc_all[:, head:head + 1]) < 1.0

            @pl.when(tail_zero)
            def _():
                zblk = jnp.zeros((d, 1024), jnp.float32)
                for j in range(head * _CHUNK, t_num, 1024):
                    flt_ref[:, j:j + 1024] = zblk

            @pl.when(jnp.logical_not(tail_zero))
            def _():
                for c in range(head, nc):
                    chunk_mask_fast(c)

    @pl.when(jnp.logical_not(fast))
    def _():
        hi = jnp.max(s, axis=1, keepdims=True)
        hik = jax.lax.bitcast_convert_type(hi, jnp.int32)
        lok = jnp.zeros_like(hik)

        def body(_, carry):
            lo, h = carry
            mid = (lo + h + 1) >> 1
            midf = jax.lax.bitcast_convert_type(mid, jnp.float32)
            cnt = jnp.sum((s >= midf).astype(jnp.int32), axis=1,
                          keepdims=True)
            ok = cnt >= _TOPK
            return jnp.where(ok, mid, lo), jnp.where(ok, h, mid - 1)

        lok, _hik = jax.lax.fori_loop(0, 31, body, (lok, hik))
        t = jax.lax.bitcast_convert_type(lok, jnp.float32)
        eqb = (s == t).astype(jnp.bfloat16)
        bfull = (jax.lax.broadcasted_iota(jnp.int32, (t_num, nc), 0)
                 // _CHUNK
                 == jax.lax.broadcasted_iota(jnp.int32, (t_num, nc), 1)
                 ).astype(jnp.bfloat16)
        tot = jax.lax.dot_general(
            eqb, bfull, (((1,), (0,)), ((), ())),
            preferred_element_type=jnp.float32)
        gt_cnt = jnp.sum((s > t).astype(jnp.float32), axis=1, keepdims=True)
        pre = jax.lax.dot_general(
            tot.astype(jnp.bfloat16), upre_ref[...],
            (((1,), (0,)), ((), ())), preferred_element_type=jnp.float32)
        needc_all = (_TOPK - gt_cnt) - pre

        for c in range(nc):
            sl = s[:, c * _CHUNK:(c + 1) * _CHUNK]
            eqc = (sl == t)
            pref = jax.lax.dot_general(
                eqc.astype(jnp.bfloat16), tri, (((1,), (0,)), ((), ())),
                preferred_element_type=jnp.float32)
            keep = (sl > t) | (eqc & (pref <= needc_all[:, c:c + 1]))
            flt_ref[:, c * _CHUNK:(c + 1) * _CHUNK] = jnp.where(
                keep, sl, jnp.float32(0.0))


def kernel(H_d, H_t, W1, W2):
    d_num, d_dim = H_d.shape
    t_num, t_dim = H_t.shape
    units = W1.shape[1]
    nc = t_num // _CHUNK
    blk = min(1024, t_num)
    Ht = pl.pallas_call(
        _proj_kernel,
        grid=(t_num // blk,),
        in_specs=[
            pl.BlockSpec((blk, t_dim), lambda i: (i, 0)),
            pl.BlockSpec((t_dim, units), lambda i: (0, 0)),
        ],
        out_specs=pl.BlockSpec((blk, units), lambda i: (i, 0)),
        out_shape=jax.ShapeDtypeStruct((t_num, units), jnp.float32),
        compiler_params=pltpu.CompilerParams(
            dimension_semantics=("parallel",)),
    )(H_t, W2)
    r256 = jnp.arange(_CHUNK, dtype=jnp.int32)
    tri = (r256[:, None] <= r256[None, :]).astype(jnp.bfloat16)
    rnc = jnp.arange(nc, dtype=jnp.int32)
    upre = (rnc[:, None] < rnc[None, :]).astype(jnp.bfloat16)

    res, flt = pl.pallas_call(
        _simtopk_kernel,
        grid=(d_num // _DBLK,),
        in_specs=[
            pl.BlockSpec((_DBLK, d_dim), lambda i: (i, 0)),
            pl.BlockSpec((t_num, units), lambda i: (0, 0)),
            pl.BlockSpec((d_dim, units), lambda i: (0, 0)),
            pl.BlockSpec((_CHUNK, _CHUNK), lambda i: (0, 0)),
            pl.BlockSpec((nc, nc), lambda i: (0, 0)),
        ],
        out_specs=[
            pl.BlockSpec((_DBLK, t_num), lambda i: (i, 0)),
            pl.BlockSpec((_DBLK, t_num), lambda i: (i, 0)),
        ],
        out_shape=[
            jax.ShapeDtypeStruct((d_num, t_num), jnp.float32),
            jax.ShapeDtypeStruct((d_num, t_num), jnp.float32),
        ],
        compiler_params=pltpu.CompilerParams(
            dimension_semantics=("arbitrary",),
            vmem_limit_bytes=64 * 1024 * 1024),
    )(H_d, Ht, W1, tri, upre)
    return res, flt


# speedup vs baseline: 1.0871x; 1.0008x over previous
import jax
import jax.numpy as jnp
from jax.experimental import pallas as pl
from jax.experimental.pallas import tpu as pltpu

_TOPK = 32
_DBLK = 256
_CHUNK = 256
_HEAD = 4


def _proj_kernel(x_ref, w_ref, o_ref):
    o_ref[...] = jnp.dot(x_ref[...], w_ref[...],
                         preferred_element_type=jnp.float32)


def _simtopk_kernel(hd_ref, htp_ref, w1_ref, tri_ref,
                    upre_ref, res_ref, flt_ref):
    hd = jnp.dot(hd_ref[...], w1_ref[...],
                 preferred_element_type=jnp.float32)
    logits = jax.lax.dot_general(
        hd, htp_ref[...], (((1,), (1,)), ((), ())),
        preferred_element_type=jnp.float32)
    s = jax.nn.sigmoid(logits)
    res_ref[...] = s
    d, t_num = s.shape
    nc = t_num // _CHUNK
    head = min(_HEAD, nc)
    one = jnp.float32(1.0)
    tri = tri_ref[...]

    _TS = 2048
    tot1 = jnp.zeros((d, nc), jnp.float32)
    for j in range(0, t_num, _TS):
        w = min(_TS, t_num - j)
        eqs = (s[:, j:j + w] == one).astype(jnp.bfloat16)
        bsl = (jax.lax.broadcasted_iota(jnp.int32, (w, nc), 0) // _CHUNK
               + j // _CHUNK
               == jax.lax.broadcasted_iota(jnp.int32, (w, nc), 1)
               ).astype(jnp.bfloat16)
        tot1 = tot1 + jax.lax.dot_general(
            eqs, bsl, (((1,), (0,)), ((), ())),
            preferred_element_type=jnp.float32)
    cnt1 = jnp.sum(tot1, axis=1, keepdims=True)
    fast = jnp.all(cnt1 >= _TOPK)

    @pl.when(fast)
    def _():
        pre = jax.lax.dot_general(
            tot1.astype(jnp.bfloat16), upre_ref[...],
            (((1,), (0,)), ((), ())), preferred_element_type=jnp.float32)
        needc_all = jnp.float32(_TOPK) - pre

        def chunk_mask_fast(c):
            sl = s[:, c * _CHUNK:(c + 1) * _CHUNK]
            eqc = (sl == one)
            pref = jax.lax.dot_general(
                eqc.astype(jnp.bfloat16), tri, (((1,), (0,)), ((), ())),
                preferred_element_type=jnp.float32)
            keep = eqc & (pref <= needc_all[:, c:c + 1])
            flt_ref[:, c * _CHUNK:(c + 1) * _CHUNK] = jnp.where(
                keep, sl, jnp.float32(0.0))

        for c in range(head):
            chunk_mask_fast(c)

        if head < nc:
            tail_zero = jnp.max(needc_all[:, head:head + 1]) < 1.0

            @pl.when(tail_zero)
            def _():
                flt_ref[:, head * _CHUNK:] = jnp.zeros(
                    (d, t_num - head * _CHUNK), jnp.float32)

            @pl.when(jnp.logical_not(tail_zero))
            def _():
                for c in range(head, nc):
                    chunk_mask_fast(c)

    @pl.when(jnp.logical_not(fast))
    def _():
        hi = jnp.max(s, axis=1, keepdims=True)
        hik = jax.lax.bitcast_convert_type(hi, jnp.int32)
        lok = jnp.zeros_like(hik)

        def body(_, carry):
            lo, h = carry
            mid = (lo + h + 1) >> 1
            midf = jax.lax.bitcast_convert_type(mid, jnp.float32)
            cnt = jnp.sum((s >= midf).astype(jnp.int32), axis=1,
                          keepdims=True)
            ok = cnt >= _TOPK
            return jnp.where(ok, mid, lo), jnp.where(ok, h, mid - 1)

        lok, _hik = jax.lax.fori_loop(0, 31, body, (lok, hik))
        t = jax.lax.bitcast_convert_type(lok, jnp.float32)
        eqb = (s == t).astype(jnp.bfloat16)
        bfull = (jax.lax.broadcasted_iota(jnp.int32, (t_num, nc), 0)
                 // _CHUNK
                 == jax.lax.broadcasted_iota(jnp.int32, (t_num, nc), 1)
                 ).astype(jnp.bfloat16)
        tot = jax.lax.dot_general(
            eqb, bfull, (((1,), (0,)), ((), ())),
            preferred_element_type=jnp.float32)
        gt_cnt = jnp.sum((s > t).astype(jnp.float32), axis=1, keepdims=True)
        pre = jax.lax.dot_general(
            tot.astype(jnp.bfloat16), upre_ref[...],
            (((1,), (0,)), ((), ())), preferred_element_type=jnp.float32)
        needc_all = (_TOPK - gt_cnt) - pre

        for c in range(nc):
            sl = s[:, c * _CHUNK:(c + 1) * _CHUNK]
            eqc = (sl == t)
            pref = jax.lax.dot_general(
                eqc.astype(jnp.bfloat16), tri, (((1,), (0,)), ((), ())),
                preferred_element_type=jnp.float32)
            keep = (sl > t) | (eqc & (pref <= needc_all[:, c:c + 1]))
            flt_ref[:, c * _CHUNK:(c + 1) * _CHUNK] = jnp.where(
                keep, sl, jnp.float32(0.0))


def kernel(H_d, H_t, W1, W2):
    d_num, d_dim = H_d.shape
    t_num, t_dim = H_t.shape
    units = W1.shape[1]
    nc = t_num // _CHUNK
    blk = min(1024, t_num)
    Ht = pl.pallas_call(
        _proj_kernel,
        grid=(t_num // blk,),
        in_specs=[
            pl.BlockSpec((blk, t_dim), lambda i: (i, 0)),
            pl.BlockSpec((t_dim, units), lambda i: (0, 0)),
        ],
        out_specs=pl.BlockSpec((blk, units), lambda i: (i, 0)),
        out_shape=jax.ShapeDtypeStruct((t_num, units), jnp.float32),
        compiler_params=pltpu.CompilerParams(
            dimension_semantics=("parallel",)),
    )(H_t, W2)
    r256 = jnp.arange(_CHUNK, dtype=jnp.int32)
    tri = (r256[:, None] <= r256[None, :]).astype(jnp.bfloat16)
    rnc = jnp.arange(nc, dtype=jnp.int32)
    upre = (rnc[:, None] < rnc[None, :]).astype(jnp.bfloat16)

    res, flt = pl.pallas_call(
        _simtopk_kernel,
        grid=(d_num // _DBLK,),
        in_specs=[
            pl.BlockSpec((_DBLK, d_dim), lambda i: (i, 0)),
            pl.BlockSpec((t_num, units), lambda i: (0, 0)),
            pl.BlockSpec((d_dim, units), lambda i: (0, 0)),
            pl.BlockSpec((_CHUNK, _CHUNK), lambda i: (0, 0)),
            pl.BlockSpec((nc, nc), lambda i: (0, 0)),
        ],
        out_specs=[
            pl.BlockSpec((_DBLK, t_num), lambda i: (i, 0)),
            pl.BlockSpec((_DBLK, t_num), lambda i: (i, 0)),
        ],
        out_shape=[
            jax.ShapeDtypeStruct((d_num, t_num), jnp.float32),
            jax.ShapeDtypeStruct((d_num, t_num), jnp.float32),
        ],
        compiler_params=pltpu.CompilerParams(
            dimension_semantics=("arbitrary",),
            vmem_limit_bytes=64 * 1024 * 1024),
    )(H_d, Ht, W1, tri, upre)
    return res, flt


# speedup vs baseline: 1.1958x; 1.1000x over previous
import jax
import jax.numpy as jnp
from jax.experimental import pallas as pl
from jax.experimental.pallas import tpu as pltpu

_TOPK = 32
_DBLK = 256
_CHUNK = 256
_HEAD = 4


def _proj_kernel(x_ref, w_ref, o_ref):
    o_ref[...] = jnp.dot(x_ref[...], w_ref[...],
                         preferred_element_type=jnp.float32)


def _simtopk_kernel(hd_ref, ht_ref, w1_ref, w2_ref, tri_ref,
                    upre_ref, res_ref, flt_ref, htp_ref):
    @pl.when(pl.program_id(0) == 0)
    def _():
        htp_ref[...] = jnp.dot(ht_ref[...], w2_ref[...],
                               preferred_element_type=jnp.float32)

    hd = jnp.dot(hd_ref[...], w1_ref[...],
                 preferred_element_type=jnp.float32)
    logits = jax.lax.dot_general(
        hd, htp_ref[...], (((1,), (1,)), ((), ())),
        preferred_element_type=jnp.float32)
    res_ref[...] = jax.nn.sigmoid(logits)
    d, t_num = res_ref.shape
    nc = t_num // _CHUNK
    head = min(_HEAD, nc)
    one = jnp.float32(1.0)
    tri = tri_ref[...]

    _TS = 2048
    tot1 = jnp.zeros((d, nc), jnp.float32)
    for j in range(0, t_num, _TS):
        w = min(_TS, t_num - j)
        eqs = (res_ref[:, j:j + w] == one).astype(jnp.bfloat16)
        bsl = (jax.lax.broadcasted_iota(jnp.int32, (w, nc), 0) // _CHUNK
               + j // _CHUNK
               == jax.lax.broadcasted_iota(jnp.int32, (w, nc), 1)
               ).astype(jnp.bfloat16)
        tot1 = tot1 + jax.lax.dot_general(
            eqs, bsl, (((1,), (0,)), ((), ())),
            preferred_element_type=jnp.float32)
    cnt1 = jnp.sum(tot1, axis=1, keepdims=True)
    fast = jnp.all(cnt1 >= _TOPK)

    @pl.when(fast)
    def _():
        pre = jax.lax.dot_general(
            tot1.astype(jnp.bfloat16), upre_ref[...],
            (((1,), (0,)), ((), ())), preferred_element_type=jnp.float32)
        needc_all = jnp.float32(_TOPK) - pre

        def chunk_mask_fast(c):
            sl = res_ref[:, c * _CHUNK:(c + 1) * _CHUNK]
            eqc = (sl == one)
            pref = jax.lax.dot_general(
                eqc.astype(jnp.bfloat16), tri, (((1,), (0,)), ((), ())),
                preferred_element_type=jnp.float32)
            keep = eqc & (pref <= needc_all[:, c:c + 1])
            flt_ref[:, c * _CHUNK:(c + 1) * _CHUNK] = jnp.where(
                keep, sl, jnp.float32(0.0))

        for c in range(head):
            chunk_mask_fast(c)

        if head < nc:
            tail_zero = jnp.max(needc_all[:, head:head + 1]) < 1.0

            @pl.when(tail_zero)
            def _():
                flt_ref[:, head * _CHUNK:] = jnp.zeros(
                    (d, t_num - head * _CHUNK), jnp.float32)

            @pl.when(jnp.logical_not(tail_zero))
            def _():
                for c in range(head, nc):
                    chunk_mask_fast(c)

    @pl.when(jnp.logical_not(fast))
    def _():
        hi = jnp.max(res_ref[...], axis=1, keepdims=True)
        hik = jax.lax.bitcast_convert_type(hi, jnp.int32)
        lok = jnp.zeros_like(hik)

        def body(_, carry):
            lo, h = carry
            mid = (lo + h + 1) >> 1
            midf = jax.lax.bitcast_convert_type(mid, jnp.float32)
            cnt = jnp.sum((res_ref[...] >= midf).astype(jnp.int32),
                          axis=1, keepdims=True)
            ok = cnt >= _TOPK
            return jnp.where(ok, mid, lo), jnp.where(ok, h, mid - 1)

        lok, _hik = jax.lax.fori_loop(0, 31, body, (lok, hik))
        t = jax.lax.bitcast_convert_type(lok, jnp.float32)
        eqb = (res_ref[...] == t).astype(jnp.bfloat16)
        bfull = (jax.lax.broadcasted_iota(jnp.int32, (t_num, nc), 0)
                 // _CHUNK
                 == jax.lax.broadcasted_iota(jnp.int32, (t_num, nc), 1)
                 ).astype(jnp.bfloat16)
        tot = jax.lax.dot_general(
            eqb, bfull, (((1,), (0,)), ((), ())),
            preferred_element_type=jnp.float32)
        gt_cnt = jnp.sum((res_ref[...] > t).astype(jnp.float32),
                           axis=1, keepdims=True)
        pre = jax.lax.dot_general(
            tot.astype(jnp.bfloat16), upre_ref[...],
            (((1,), (0,)), ((), ())), preferred_element_type=jnp.float32)
        needc_all = (_TOPK - gt_cnt) - pre

        for c in range(nc):
            sl = res_ref[:, c * _CHUNK:(c + 1) * _CHUNK]
            eqc = (sl == t)
            pref = jax.lax.dot_general(
                eqc.astype(jnp.bfloat16), tri, (((1,), (0,)), ((), ())),
                preferred_element_type=jnp.float32)
            keep = (sl > t) | (eqc & (pref <= needc_all[:, c:c + 1]))
            flt_ref[:, c * _CHUNK:(c + 1) * _CHUNK] = jnp.where(
                keep, sl, jnp.float32(0.0))


def kernel(H_d, H_t, W1, W2):
    d_num, d_dim = H_d.shape
    t_num, t_dim = H_t.shape
    units = W1.shape[1]
    nc = t_num // _CHUNK
    r256 = jnp.arange(_CHUNK, dtype=jnp.int32)
    tri = (r256[:, None] <= r256[None, :]).astype(jnp.bfloat16)
    rnc = jnp.arange(nc, dtype=jnp.int32)
    upre = (rnc[:, None] < rnc[None, :]).astype(jnp.bfloat16)

    res, flt = pl.pallas_call(
        _simtopk_kernel,
        grid=(d_num // _DBLK,),
        in_specs=[
            pl.BlockSpec((_DBLK, d_dim), lambda i: (i, 0)),
            pl.BlockSpec((t_num, t_dim), lambda i: (0, 0)),
            pl.BlockSpec((d_dim, units), lambda i: (0, 0)),
            pl.BlockSpec((t_dim, units), lambda i: (0, 0)),
            pl.BlockSpec((_CHUNK, _CHUNK), lambda i: (0, 0)),
            pl.BlockSpec((nc, nc), lambda i: (0, 0)),
        ],
        out_specs=[
            pl.BlockSpec((_DBLK, t_num), lambda i: (i, 0)),
            pl.BlockSpec((_DBLK, t_num), lambda i: (i, 0)),
        ],
        out_shape=[
            jax.ShapeDtypeStruct((d_num, t_num), jnp.float32),
            jax.ShapeDtypeStruct((d_num, t_num), jnp.float32),
        ],
        scratch_shapes=[
            pltpu.VMEM((t_num, units), jnp.float32),
        ],
        compiler_params=pltpu.CompilerParams(
            dimension_semantics=("arbitrary",),
            vmem_limit_bytes=64 * 1024 * 1024),
    )(H_d, H_t, W1, W2, tri, upre)
    return res, flt


# speedup vs baseline: 1.2020x; 1.0051x over previous
import jax
import jax.numpy as jnp
from jax.experimental import pallas as pl
from jax.experimental.pallas import tpu as pltpu

_TOPK = 32
_DBLK = 256
_CHUNK = 256
_HEAD = 4


def _simtopk_kernel(hd_ref, ht_ref, w1_ref, w2_ref, tri_ref,
                    upre_ref, res_ref, flt_ref, htp_ref):
    @pl.when(pl.program_id(0) == 0)
    def _():
        htp_ref[...] = jnp.dot(ht_ref[...], w2_ref[...],
                               preferred_element_type=jnp.float32)

    hd = jnp.dot(hd_ref[...], w1_ref[...],
                 preferred_element_type=jnp.float32)
    logits = jax.lax.dot_general(
        hd, htp_ref[...], (((1,), (1,)), ((), ())),
        preferred_element_type=jnp.float32)
    res_ref[...] = jax.nn.sigmoid(logits)
    d, t_num = res_ref.shape
    nc = t_num // _CHUNK
    head = min(_HEAD, nc)
    one = jnp.float32(1.0)
    tri = tri_ref[...]

    _TS = 2048
    tot1 = jnp.zeros((d, nc), jnp.float32)
    for j in range(0, t_num, _TS):
        w = min(_TS, t_num - j)
        eqs = (res_ref[:, j:j + w] == one).astype(jnp.bfloat16)
        bsl = (jax.lax.broadcasted_iota(jnp.int32, (w, nc), 0) // _CHUNK
               + j // _CHUNK
               == jax.lax.broadcasted_iota(jnp.int32, (w, nc), 1)
               ).astype(jnp.bfloat16)
        tot1 = tot1 + jax.lax.dot_general(
            eqs, bsl, (((1,), (0,)), ((), ())),
            preferred_element_type=jnp.float32)
    cnt1 = jnp.sum(tot1, axis=1, keepdims=True)
    fast = jnp.all(cnt1 >= _TOPK)

    @pl.when(fast)
    def _():
        pre = jax.lax.dot_general(
            tot1.astype(jnp.bfloat16), upre_ref[...],
            (((1,), (0,)), ((), ())), preferred_element_type=jnp.float32)
        needc_all = jnp.float32(_TOPK) - pre

        def chunk_mask_fast(c):
            sl = res_ref[:, c * _CHUNK:(c + 1) * _CHUNK]
            eqc = (sl == one)
            pref = jax.lax.dot_general(
                eqc.astype(jnp.bfloat16), tri, (((1,), (0,)), ((), ())),
                preferred_element_type=jnp.float32)
            keep = eqc & (pref <= needc_all[:, c:c + 1])
            flt_ref[:, c * _CHUNK:(c + 1) * _CHUNK] = jnp.where(
                keep, sl, jnp.float32(0.0))

        for c in range(head):
            chunk_mask_fast(c)

        if head < nc:
            tail_zero = jnp.max(needc_all[:, head:head + 1]) < 1.0

            @pl.when(tail_zero)
            def _():
                flt_ref[:, head * _CHUNK:] = jnp.zeros(
                    (d, t_num - head * _CHUNK), jnp.float32)

            @pl.when(jnp.logical_not(tail_zero))
            def _():
                for c in range(head, nc):
                    chunk_mask_fast(c)

    @pl.when(jnp.logical_not(fast))
    def _():
        hi = jnp.max(res_ref[...], axis=1, keepdims=True)
        hik = jax.lax.bitcast_convert_type(hi, jnp.int32)
        lok = jnp.zeros_like(hik)

        def body(_, carry):
            lo, h = carry
            mid = (lo + h + 1) >> 1
            midf = jax.lax.bitcast_convert_type(mid, jnp.float32)
            cnt = jnp.sum((res_ref[...] >= midf).astype(jnp.int32),
                          axis=1, keepdims=True)
            ok = cnt >= _TOPK
            return jnp.where(ok, mid, lo), jnp.where(ok, h, mid - 1)

        lok, _hik = jax.lax.fori_loop(0, 31, body, (lok, hik))
        t = jax.lax.bitcast_convert_type(lok, jnp.float32)
        eqb = (res_ref[...] == t).astype(jnp.bfloat16)
        bfull = (jax.lax.broadcasted_iota(jnp.int32, (t_num, nc), 0)
                 // _CHUNK
                 == jax.lax.broadcasted_iota(jnp.int32, (t_num, nc), 1)
                 ).astype(jnp.bfloat16)
        tot = jax.lax.dot_general(
            eqb, bfull, (((1,), (0,)), ((), ())),
            preferred_element_type=jnp.float32)
        gt_cnt = jnp.sum((res_ref[...] > t).astype(jnp.float32),
                           axis=1, keepdims=True)
        pre = jax.lax.dot_general(
            tot.astype(jnp.bfloat16), upre_ref[...],
            (((1,), (0,)), ((), ())), preferred_element_type=jnp.float32)
        needc_all = (_TOPK - gt_cnt) - pre

        for c in range(nc):
            sl = res_ref[:, c * _CHUNK:(c + 1) * _CHUNK]
            eqc = (sl == t)
            pref = jax.lax.dot_general(
                eqc.astype(jnp.bfloat16), tri, (((1,), (0,)), ((), ())),
                preferred_element_type=jnp.float32)
            keep = (sl > t) | (eqc & (pref <= needc_all[:, c:c + 1]))
            flt_ref[:, c * _CHUNK:(c + 1) * _CHUNK] = jnp.where(
                keep, sl, jnp.float32(0.0))


def kernel(H_d, H_t, W1, W2):
    d_num, d_dim = H_d.shape
    t_num, t_dim = H_t.shape
    units = W1.shape[1]
    nc = t_num // _CHUNK
    r256 = jnp.arange(_CHUNK, dtype=jnp.int32)
    tri = (r256[:, None] <= r256[None, :]).astype(jnp.bfloat16)
    rnc = jnp.arange(nc, dtype=jnp.int32)
    upre = (rnc[:, None] < rnc[None, :]).astype(jnp.bfloat16)

    res, flt = pl.pallas_call(
        _simtopk_kernel,
        grid=(d_num // _DBLK,),
        in_specs=[
            pl.BlockSpec((_DBLK, d_dim), lambda i: (i, 0)),
            pl.BlockSpec((t_num, t_dim), lambda i: (0, 0)),
            pl.BlockSpec((d_dim, units), lambda i: (0, 0)),
            pl.BlockSpec((t_dim, units), lambda i: (0, 0)),
            pl.BlockSpec((_CHUNK, _CHUNK), lambda i: (0, 0)),
            pl.BlockSpec((nc, nc), lambda i: (0, 0)),
        ],
        out_specs=[
            pl.BlockSpec((_DBLK, t_num), lambda i: (i, 0)),
            pl.BlockSpec((_DBLK, t_num), lambda i: (i, 0)),
        ],
        out_shape=[
            jax.ShapeDtypeStruct((d_num, t_num), jnp.float32),
            jax.ShapeDtypeStruct((d_num, t_num), jnp.float32),
        ],
        scratch_shapes=[
            pltpu.VMEM((t_num, units), jnp.float32),
        ],
        compiler_params=pltpu.CompilerParams(
            dimension_semantics=("arbitrary",),
            vmem_limit_bytes=64 * 1024 * 1024),
    )(H_d, H_t, W1, W2, tri, upre)
    return res, flt
